# baseline (speedup 1.0000x reference)
import functools

import jax
import jax.numpy as jnp
from jax import lax
from jax.experimental import pallas as pl
from jax.experimental.pallas import tpu as pltpu
from jax.experimental.pallas import tpu_sc as plsc

TXT_LEN = 512
IMG_LEN = 256
TOT_LEN = TXT_LEN + IMG_LEN
NUM_HEADS = 16
BATCH = 2
R = 64
NW = 32


def _bucket_luts(anchor):
    eps = jnp.finfo(jnp.float32).eps
    d = jnp.arange(-(TXT_LEN - 1), TXT_LEN, dtype=jnp.int32) + anchor
    n = -d
    r0 = (n < 0).astype(jnp.int32) * 16
    n = jnp.abs(n)
    vl = 8 + (jnp.log(n.astype(jnp.float32) / 8 + eps) / jnp.log(128.0 / 8) * 8).astype(jnp.int32)
    vl = jnp.minimum(vl, 15)
    lut_txt = r0 + jnp.where(n < 8, n, vl)
    lut_txt = jnp.concatenate([lut_txt, jnp.zeros((1,), jnp.int32)])

    dy = (jnp.arange(-15, 16, dtype=jnp.int32) + anchor)[:, None]
    dx = (jnp.arange(-15, 16, dtype=jnp.int32) + anchor)[None, :]
    nx, ny = -dx, -dy
    r = (jnp.logical_and(nx <= 0, ny < 0).astype(jnp.int32) * 192
         + jnp.logical_and(nx < 0, ny >= 0).astype(jnp.int32) * 128
         + jnp.logical_and(nx > 0, ny <= 0).astype(jnp.int32) * 64)
    nx, ny = jnp.abs(nx), jnp.abs(ny)
    vx = 4 + (jnp.log(nx.astype(jnp.float32) / 4 + eps) / jnp.log(20.0 / 4) * 4).astype(jnp.int32)
    vx = jnp.minimum(vx, 7)
    vy = 4 + (jnp.log(ny.astype(jnp.float32) / 4 + eps) / jnp.log(20.0 / 4) * 4).astype(jnp.int32)
    vy = jnp.minimum(vy, 7)
    xx = jnp.where(nx < 4, nx, vx)
    yy = jnp.where(ny < 4, ny, vy)
    lut_img = (r + xx + 8 * yy).reshape(-1)
    lut_img = jnp.concatenate([lut_img, jnp.zeros((63,), jnp.int32)])
    return lut_txt, lut_img


def _sc_fill(txt_pos, img_pos, rel_emb, img_rel_emb, lut_txt, lut_img):
    mesh = plsc.VectorSubcoreMesh(core_axis_name="c", subcore_axis_name="s")

    @functools.partial(
        pl.kernel,
        mesh=mesh,
        compiler_params=pltpu.CompilerParams(needs_layout_passes=False),
        out_type=jax.ShapeDtypeStruct((BATCH, NUM_HEADS, TOT_LEN, TOT_LEN), jnp.float32),
        scratch_types=[
            pltpu.VMEM((TXT_LEN,), jnp.int32),
            pltpu.VMEM((IMG_LEN,), jnp.int32),
            pltpu.VMEM((32,), jnp.float32),
            pltpu.VMEM((256,), jnp.float32),
            pltpu.VMEM((1024,), jnp.int32),
            pltpu.VMEM((1024,), jnp.int32),
            pltpu.VMEM((1024,), jnp.float32),
            pltpu.VMEM((1024,), jnp.float32),
            pltpu.VMEM((2 * R, TOT_LEN), jnp.float32),
            pltpu.SemaphoreType.DMA,
            pltpu.SemaphoreType.DMA,
        ],
    )
    def body(txt_pos_hbm, img_pos_hbm, rel_hbm, img_rel_hbm,
             lut_txt_hbm, lut_img_hbm, out_hbm,
             tpos_v, ipos_v, trow_v, irow_v, ltxt_v, limg_v,
             ttab_v, itab_v, buf_v, sem, sem_in):
        wid = lax.axis_index("s") * 2 + lax.axis_index("c")
        b = wid // NUM_HEADS
        h = wid % NUM_HEADS

        copies = [
            pltpu.async_copy(txt_pos_hbm.at[b], tpos_v, sem_in),
            pltpu.async_copy(img_pos_hbm.at[b], ipos_v, sem_in),
            pltpu.async_copy(rel_hbm.at[h], trow_v, sem_in),
            pltpu.async_copy(img_rel_hbm.at[h], irow_v, sem_in),
            pltpu.async_copy(lut_txt_hbm, ltxt_v, sem_in),
            pltpu.async_copy(lut_img_hbm, limg_v, sem_in),
        ]
        for cp in copies:
            cp.wait()

        def bf16_round(i, ref_v):
            v = ref_v[pl.ds(i * 16, 16)]
            u = plsc.bitcast(v, jnp.uint32)
            bias = jnp.uint32(0x7FFF) + ((u >> 16) & jnp.uint32(1))
            u = (u + bias) & jnp.uint32(0xFFFF0000)
            ref_v[pl.ds(i * 16, 16)] = plsc.bitcast(u, jnp.float32)

        for i in range(2):
            bf16_round(i, trow_v)
        for i in range(16):
            bf16_round(i, irow_v)

        def build(tab_v, lut_v, row_v, quarter):
            idxs = [lut_v[pl.ds((quarter * 16 + i) * 16, 16)] for i in range(16)]
            vals = [plsc.load_gather(row_v, [idx]) for idx in idxs]
            for i in range(16):
                tab_v[pl.ds((quarter * 16 + i) * 16, 16)] = vals[i]

        for q in range(4):
            build(ttab_v, ltxt_v, trow_v, q)
        for q in range(4):
            build(itab_v, limg_v, irow_v, q)

        def conv(i, _):
            p = ipos_v[pl.ds(i * 16, 16)]
            ipos_v[pl.ds(i * 16, 16)] = (p & 15) + 31 * (p >> 4)
            return _
        lax.fori_loop(0, IMG_LEN // 16, conv, None)

        zeros16 = jnp.zeros((16,), jnp.float32)

        def wait_one():
            pltpu.make_async_copy(
                buf_v.at[pl.ds(0, R)],
                out_hbm.at[b, h, pl.ds(0, R)], sem).wait()

        tpj = [tpos_v[pl.ds(c * 16, 16)] + (TXT_LEN - 1) for c in range(TXT_LEN // 16)]

        def txt_block(blk, _):
            off = (blk & 1) * R
            i0 = blk * R

            @pl.when(blk >= 2)
            def _w():
                wait_one()

            def group(g, _):
                svec = tpos_v[pl.ds(i0 + g * 16, 16)]
                row0 = off + g * 16
                for r2 in range(16):
                    s = svec[r2]
                    for half in range(2):
                        vals = [plsc.load_gather(ttab_v, [tpj[half * 16 + c] - s])
                                for c in range(16)]
                        for c in range(16):
                            buf_v[row0 + r2, pl.ds((half * 16 + c) * 16, 16)] = vals[c]
                    for c in range(IMG_LEN // 16):
                        buf_v[row0 + r2, pl.ds(TXT_LEN + c * 16, 16)] = zeros16
                return _
            lax.fori_loop(0, R // 16, group, None)
            pltpu.async_copy(buf_v.at[pl.ds(off, R)],
                             out_hbm.at[b, h, pl.ds(i0, R)], sem)
            return _
        lax.fori_loop(0, TXT_LEN // R, txt_block, None)

        ipj = [ipos_v[pl.ds(c * 16, 16)] + 480 for c in range(IMG_LEN // 16)]

        def img_block(blk, _):
            off = (blk & 1) * R
            m0 = blk * R

            wait_one()

            def group(g, _):
                svec = ipos_v[pl.ds(m0 + g * 16, 16)]
                row0 = off + g * 16
                for r2 in range(16):
                    s = svec[r2]
                    vals = [plsc.load_gather(itab_v, [ipj[c] - s])
                            for c in range(16)]
                    for c in range(16):
                        buf_v[row0 + r2, pl.ds(TXT_LEN + c * 16, 16)] = vals[c]
                    for c in range(TXT_LEN // 16):
                        buf_v[row0 + r2, pl.ds(c * 16, 16)] = zeros16
                return _
            lax.fori_loop(0, R // 16, group, None)
            pltpu.async_copy(
                buf_v.at[pl.ds(off, R)],
                out_hbm.at[b, h, pl.ds(TXT_LEN + m0, R)], sem)
            return _
        lax.fori_loop(0, IMG_LEN // R, img_block, None)
        wait_one()
        wait_one()

    return body(txt_pos, img_pos, rel_emb, img_rel_emb, lut_txt, lut_img)


def kernel(txt_position_ids, img_position_ids, rel_embedding, image_rel_embedding):
    anchor = (txt_position_ids[0, 0] * 0).astype(jnp.int32)
    lut_txt, lut_img = _bucket_luts(anchor)
    return _sc_fill(
        txt_position_ids.astype(jnp.int32),
        img_position_ids.astype(jnp.int32),
        rel_embedding.astype(jnp.float32),
        image_rel_embedding.astype(jnp.float32),
        lut_txt, lut_img)


# speedup vs baseline: 1.1278x; 1.1278x over previous
import functools

import jax
import jax.numpy as jnp
from jax import lax
from jax.experimental import pallas as pl
from jax.experimental.pallas import tpu as pltpu
from jax.experimental.pallas import tpu_sc as plsc

TXT_LEN = 512
IMG_LEN = 256
TOT_LEN = TXT_LEN + IMG_LEN
NUM_HEADS = 16
BATCH = 2
R = 64
NW = 32


def _bucket_luts(anchor):
    eps = jnp.finfo(jnp.float32).eps
    d = jnp.arange(-(TXT_LEN - 1), TXT_LEN, dtype=jnp.int32) + anchor
    n = -d
    r0 = (n < 0).astype(jnp.int32) * 16
    n = jnp.abs(n)
    vl = 8 + (jnp.log(n.astype(jnp.float32) / 8 + eps) / jnp.log(128.0 / 8) * 8).astype(jnp.int32)
    vl = jnp.minimum(vl, 15)
    lut_txt = r0 + jnp.where(n < 8, n, vl)
    lut_txt = jnp.concatenate([lut_txt, jnp.zeros((1,), jnp.int32)])

    dy = (jnp.arange(-15, 16, dtype=jnp.int32) + anchor)[:, None]
    dx = (jnp.arange(-15, 16, dtype=jnp.int32) + anchor)[None, :]
    nx, ny = -dx, -dy
    r = (jnp.logical_and(nx <= 0, ny < 0).astype(jnp.int32) * 192
         + jnp.logical_and(nx < 0, ny >= 0).astype(jnp.int32) * 128
         + jnp.logical_and(nx > 0, ny <= 0).astype(jnp.int32) * 64)
    nx, ny = jnp.abs(nx), jnp.abs(ny)
    vx = 4 + (jnp.log(nx.astype(jnp.float32) / 4 + eps) / jnp.log(20.0 / 4) * 4).astype(jnp.int32)
    vx = jnp.minimum(vx, 7)
    vy = 4 + (jnp.log(ny.astype(jnp.float32) / 4 + eps) / jnp.log(20.0 / 4) * 4).astype(jnp.int32)
    vy = jnp.minimum(vy, 7)
    xx = jnp.where(nx < 4, nx, vx)
    yy = jnp.where(ny < 4, ny, vy)
    lut_img = (r + xx + 8 * yy).reshape(-1)
    lut_img = jnp.concatenate([lut_img, jnp.zeros((63,), jnp.int32)])
    return lut_txt, lut_img


def _sc_fill(txt_pos, img_pos, rel_emb, img_rel_emb, lut_txt, lut_img):
    mesh = plsc.VectorSubcoreMesh(core_axis_name="c", subcore_axis_name="s")

    @functools.partial(
        pl.kernel,
        mesh=mesh,
        compiler_params=pltpu.CompilerParams(needs_layout_passes=False),
        out_type=jax.ShapeDtypeStruct((BATCH, NUM_HEADS, TOT_LEN, TOT_LEN), jnp.float32),
        scratch_types=[
            pltpu.VMEM((TXT_LEN,), jnp.int32),
            pltpu.VMEM((IMG_LEN,), jnp.int32),
            pltpu.VMEM((32,), jnp.float32),
            pltpu.VMEM((256,), jnp.float32),
            pltpu.VMEM((1024,), jnp.int32),
            pltpu.VMEM((1024,), jnp.int32),
            pltpu.VMEM((1024,), jnp.float32),
            pltpu.VMEM((1024,), jnp.float32),
            pltpu.VMEM((2 * R, TOT_LEN), jnp.float32),
            pltpu.SemaphoreType.DMA,
            pltpu.SemaphoreType.DMA,
        ],
    )
    def body(txt_pos_hbm, img_pos_hbm, rel_hbm, img_rel_hbm,
             lut_txt_hbm, lut_img_hbm, out_hbm,
             tpos_v, ipos_v, trow_v, irow_v, ltxt_v, limg_v,
             ttab_v, itab_v, buf_v, sem, sem_in):
        wid = lax.axis_index("s") * 2 + lax.axis_index("c")
        b = wid // NUM_HEADS
        h = wid % NUM_HEADS

        copies = [
            pltpu.async_copy(txt_pos_hbm.at[b], tpos_v, sem_in),
            pltpu.async_copy(img_pos_hbm.at[b], ipos_v, sem_in),
            pltpu.async_copy(rel_hbm.at[h], trow_v, sem_in),
            pltpu.async_copy(img_rel_hbm.at[h], irow_v, sem_in),
            pltpu.async_copy(lut_txt_hbm, ltxt_v, sem_in),
            pltpu.async_copy(lut_img_hbm, limg_v, sem_in),
        ]
        for cp in copies:
            cp.wait()

        def bf16_round(i, ref_v):
            v = ref_v[pl.ds(i * 16, 16)]
            u = plsc.bitcast(v, jnp.uint32)
            bias = jnp.uint32(0x7FFF) + ((u >> 16) & jnp.uint32(1))
            u = (u + bias) & jnp.uint32(0xFFFF0000)
            ref_v[pl.ds(i * 16, 16)] = plsc.bitcast(u, jnp.float32)

        for i in range(2):
            bf16_round(i, trow_v)
        for i in range(16):
            bf16_round(i, irow_v)

        def build(tab_v, lut_v, row_v, quarter):
            idxs = [lut_v[pl.ds((quarter * 16 + i) * 16, 16)] for i in range(16)]
            vals = [plsc.load_gather(row_v, [idx]) for idx in idxs]
            for i in range(16):
                tab_v[pl.ds((quarter * 16 + i) * 16, 16)] = vals[i]

        for q in range(4):
            build(ttab_v, ltxt_v, trow_v, q)
        for q in range(4):
            build(itab_v, limg_v, irow_v, q)

        def conv(i, _):
            p = ipos_v[pl.ds(i * 16, 16)]
            ipos_v[pl.ds(i * 16, 16)] = (p & 15) + 31 * (p >> 4)
            return _
        lax.fori_loop(0, IMG_LEN // 16, conv, None)

        zeros16 = jnp.zeros((16,), jnp.float32)

        def wait_one():
            pltpu.make_async_copy(
                buf_v.at[pl.ds(0, R)],
                out_hbm.at[b, h, pl.ds(0, R)], sem).wait()

        tpj = [tpos_v[pl.ds(c * 16, 16)] + (TXT_LEN - 1) for c in range(TXT_LEN // 16)]

        def txt_block(blk, _):
            off = (blk & 1) * R
            i0 = blk * R

            @pl.when(blk >= 2)
            def _w():
                wait_one()

            @pl.when(blk < 2)
            def _z():
                def zrow(r2, _):
                    for c in range(IMG_LEN // 16):
                        buf_v[off + r2, pl.ds(TXT_LEN + c * 16, 16)] = zeros16
                    return _
                lax.fori_loop(0, R, zrow, None)

            def group(g, _):
                svec = tpos_v[pl.ds(i0 + g * 16, 16)]
                row0 = off + g * 16
                for r2 in range(16):
                    s = svec[r2]
                    for half in range(2):
                        vals = [plsc.load_gather(ttab_v, [tpj[half * 16 + c] - s])
                                for c in range(16)]
                        for c in range(16):
                            buf_v[row0 + r2, pl.ds((half * 16 + c) * 16, 16)] = vals[c]
                return _
            lax.fori_loop(0, R // 16, group, None)
            pltpu.async_copy(buf_v.at[pl.ds(off, R)],
                             out_hbm.at[b, h, pl.ds(i0, R)], sem)
            return _
        lax.fori_loop(0, TXT_LEN // R, txt_block, None)

        ipj = [ipos_v[pl.ds(c * 16, 16)] + 480 for c in range(IMG_LEN // 16)]

        def img_block(blk, _):
            off = (blk & 1) * R
            m0 = blk * R

            wait_one()

            @pl.when(blk < 2)
            def _z():
                def zrow(r2, _):
                    for c in range(TXT_LEN // 16):
                        buf_v[off + r2, pl.ds(c * 16, 16)] = zeros16
                    return _
                lax.fori_loop(0, R, zrow, None)

            def group(g, _):
                svec = ipos_v[pl.ds(m0 + g * 16, 16)]
                row0 = off + g * 16
                for r2 in range(16):
                    s = svec[r2]
                    vals = [plsc.load_gather(itab_v, [ipj[c] - s])
                            for c in range(16)]
                    for c in range(16):
                        buf_v[row0 + r2, pl.ds(TXT_LEN + c * 16, 16)] = vals[c]
                return _
            lax.fori_loop(0, R // 16, group, None)
            pltpu.async_copy(
                buf_v.at[pl.ds(off, R)],
                out_hbm.at[b, h, pl.ds(TXT_LEN + m0, R)], sem)
            return _
        lax.fori_loop(0, IMG_LEN // R, img_block, None)
        wait_one()
        wait_one()

    return body(txt_pos, img_pos, rel_emb, img_rel_emb, lut_txt, lut_img)


def kernel(txt_position_ids, img_position_ids, rel_embedding, image_rel_embedding):
    anchor = (txt_position_ids[0, 0] * 0).astype(jnp.int32)
    lut_txt, lut_img = _bucket_luts(anchor)
    return _sc_fill(
        txt_position_ids.astype(jnp.int32),
        img_position_ids.astype(jnp.int32),
        rel_embedding.astype(jnp.float32),
        image_rel_embedding.astype(jnp.float32),
        lut_txt, lut_img)


# speedup vs baseline: 1.1690x; 1.0365x over previous
import functools

import jax
import jax.numpy as jnp
from jax import lax
from jax.experimental import pallas as pl
from jax.experimental.pallas import tpu as pltpu
from jax.experimental.pallas import tpu_sc as plsc

TXT_LEN = 512
IMG_LEN = 256
TOT_LEN = TXT_LEN + IMG_LEN
NUM_HEADS = 16
BATCH = 2
R = 32
NW = 32


def _bucket_luts(anchor):
    eps = jnp.finfo(jnp.float32).eps
    d = jnp.arange(-(TXT_LEN - 1), TXT_LEN, dtype=jnp.int32) + anchor
    n = -d
    r0 = (n < 0).astype(jnp.int32) * 16
    n = jnp.abs(n)
    vl = 8 + (jnp.log(n.astype(jnp.float32) / 8 + eps) / jnp.log(128.0 / 8) * 8).astype(jnp.int32)
    vl = jnp.minimum(vl, 15)
    lut_txt = r0 + jnp.where(n < 8, n, vl)
    lut_txt = jnp.concatenate([lut_txt, jnp.zeros((1,), jnp.int32)])

    dy = (jnp.arange(-15, 16, dtype=jnp.int32) + anchor)[:, None]
    dx = (jnp.arange(-15, 16, dtype=jnp.int32) + anchor)[None, :]
    nx, ny = -dx, -dy
    r = (jnp.logical_and(nx <= 0, ny < 0).astype(jnp.int32) * 192
         + jnp.logical_and(nx < 0, ny >= 0).astype(jnp.int32) * 128
         + jnp.logical_and(nx > 0, ny <= 0).astype(jnp.int32) * 64)
    nx, ny = jnp.abs(nx), jnp.abs(ny)
    vx = 4 + (jnp.log(nx.astype(jnp.float32) / 4 + eps) / jnp.log(20.0 / 4) * 4).astype(jnp.int32)
    vx = jnp.minimum(vx, 7)
    vy = 4 + (jnp.log(ny.astype(jnp.float32) / 4 + eps) / jnp.log(20.0 / 4) * 4).astype(jnp.int32)
    vy = jnp.minimum(vy, 7)
    xx = jnp.where(nx < 4, nx, vx)
    yy = jnp.where(ny < 4, ny, vy)
    lut_img = (r + xx + 8 * yy).reshape(-1)
    lut_img = jnp.concatenate([lut_img, jnp.zeros((63,), jnp.int32)])
    return lut_txt, lut_img


def _sc_fill(txt_pos, img_pos, rel_emb, img_rel_emb, lut_txt, lut_img):
    mesh = plsc.VectorSubcoreMesh(core_axis_name="c", subcore_axis_name="s")

    @functools.partial(
        pl.kernel,
        mesh=mesh,
        compiler_params=pltpu.CompilerParams(needs_layout_passes=False),
        out_type=jax.ShapeDtypeStruct((BATCH, NUM_HEADS, TOT_LEN, TOT_LEN), jnp.float32),
        scratch_types=[
            pltpu.VMEM((TXT_LEN,), jnp.int32),
            pltpu.VMEM((IMG_LEN,), jnp.int32),
            pltpu.VMEM((32,), jnp.float32),
            pltpu.VMEM((256,), jnp.float32),
            pltpu.VMEM((1024,), jnp.int32),
            pltpu.VMEM((1024,), jnp.int32),
            pltpu.VMEM((1024,), jnp.float32),
            pltpu.VMEM((1024,), jnp.float32),
            pltpu.VMEM((2 * R, TOT_LEN), jnp.float32),
            pltpu.SemaphoreType.DMA,
            pltpu.SemaphoreType.DMA,
        ],
    )
    def body(txt_pos_hbm, img_pos_hbm, rel_hbm, img_rel_hbm,
             lut_txt_hbm, lut_img_hbm, out_hbm,
             tpos_v, ipos_v, trow_v, irow_v, ltxt_v, limg_v,
             ttab_v, itab_v, buf_v, sem, sem_in):
        wid = lax.axis_index("s") * 2 + lax.axis_index("c")
        b = wid // NUM_HEADS
        h = wid % NUM_HEADS

        copies = [
            pltpu.async_copy(txt_pos_hbm.at[b], tpos_v, sem_in),
            pltpu.async_copy(img_pos_hbm.at[b], ipos_v, sem_in),
            pltpu.async_copy(rel_hbm.at[h], trow_v, sem_in),
            pltpu.async_copy(img_rel_hbm.at[h], irow_v, sem_in),
            pltpu.async_copy(lut_txt_hbm, ltxt_v, sem_in),
            pltpu.async_copy(lut_img_hbm, limg_v, sem_in),
        ]
        for cp in copies:
            cp.wait()

        def bf16_round(i, ref_v):
            v = ref_v[pl.ds(i * 16, 16)]
            u = plsc.bitcast(v, jnp.uint32)
            bias = jnp.uint32(0x7FFF) + ((u >> 16) & jnp.uint32(1))
            u = (u + bias) & jnp.uint32(0xFFFF0000)
            ref_v[pl.ds(i * 16, 16)] = plsc.bitcast(u, jnp.float32)

        for i in range(2):
            bf16_round(i, trow_v)
        for i in range(16):
            bf16_round(i, irow_v)

        def build(tab_v, lut_v, row_v, quarter):
            idxs = [lut_v[pl.ds((quarter * 16 + i) * 16, 16)] for i in range(16)]
            vals = [plsc.load_gather(row_v, [idx]) for idx in idxs]
            for i in range(16):
                tab_v[pl.ds((quarter * 16 + i) * 16, 16)] = vals[i]

        for q in range(4):
            build(ttab_v, ltxt_v, trow_v, q)
        for q in range(4):
            build(itab_v, limg_v, irow_v, q)

        def conv(i, _):
            p = ipos_v[pl.ds(i * 16, 16)]
            ipos_v[pl.ds(i * 16, 16)] = (p & 15) + 31 * (p >> 4)
            return _
        lax.fori_loop(0, IMG_LEN // 16, conv, None)

        zeros16 = jnp.zeros((16,), jnp.float32)

        def wait_one():
            pltpu.make_async_copy(
                buf_v.at[pl.ds(0, R)],
                out_hbm.at[b, h, pl.ds(0, R)], sem).wait()

        tpj = [tpos_v[pl.ds(c * 16, 16)] + (TXT_LEN - 1) for c in range(TXT_LEN // 16)]

        def txt_block(blk, _):
            off = (blk & 1) * R
            i0 = blk * R

            @pl.when(blk >= 2)
            def _w():
                wait_one()

            @pl.when(blk < 2)
            def _z():
                def zrow(r2, _):
                    for c in range(IMG_LEN // 16):
                        buf_v[off + r2, pl.ds(TXT_LEN + c * 16, 16)] = zeros16
                    return _
                lax.fori_loop(0, R, zrow, None)

            def group(g, _):
                svec = tpos_v[pl.ds(i0 + g * 16, 16)]
                row0 = off + g * 16
                for r2 in range(16):
                    s = svec[r2]
                    for half in range(2):
                        vals = [plsc.load_gather(ttab_v, [tpj[half * 16 + c] - s])
                                for c in range(16)]
                        for c in range(16):
                            buf_v[row0 + r2, pl.ds((half * 16 + c) * 16, 16)] = vals[c]
                return _
            lax.fori_loop(0, R // 16, group, None)
            pltpu.async_copy(buf_v.at[pl.ds(off, R)],
                             out_hbm.at[b, h, pl.ds(i0, R)], sem)
            return _
        lax.fori_loop(0, TXT_LEN // R, txt_block, None)

        ipj = [ipos_v[pl.ds(c * 16, 16)] + 480 for c in range(IMG_LEN // 16)]

        def img_block(blk, _):
            off = (blk & 1) * R
            m0 = blk * R

            wait_one()

            @pl.when(blk < 2)
            def _z():
                def zrow(r2, _):
                    for c in range(TXT_LEN // 16):
                        buf_v[off + r2, pl.ds(c * 16, 16)] = zeros16
                    return _
                lax.fori_loop(0, R, zrow, None)

            def group(g, _):
                svec = ipos_v[pl.ds(m0 + g * 16, 16)]
                row0 = off + g * 16
                for r2 in range(16):
                    s = svec[r2]
                    vals = [plsc.load_gather(itab_v, [ipj[c] - s])
                            for c in range(16)]
                    for c in range(16):
                        buf_v[row0 + r2, pl.ds(TXT_LEN + c * 16, 16)] = vals[c]
                return _
            lax.fori_loop(0, R // 16, group, None)
            pltpu.async_copy(
                buf_v.at[pl.ds(off, R)],
                out_hbm.at[b, h, pl.ds(TXT_LEN + m0, R)], sem)
            return _
        lax.fori_loop(0, IMG_LEN // R, img_block, None)
        wait_one()
        wait_one()

    return body(txt_pos, img_pos, rel_emb, img_rel_emb, lut_txt, lut_img)


def kernel(txt_position_ids, img_position_ids, rel_embedding, image_rel_embedding):
    anchor = (txt_position_ids[0, 0] * 0).astype(jnp.int32)
    lut_txt, lut_img = _bucket_luts(anchor)
    return _sc_fill(
        txt_position_ids.astype(jnp.int32),
        img_position_ids.astype(jnp.int32),
        rel_embedding.astype(jnp.float32),
        image_rel_embedding.astype(jnp.float32),
        lut_txt, lut_img)


# speedup vs baseline: 1.1920x; 1.0197x over previous
import functools

import jax
import jax.numpy as jnp
from jax import lax
from jax.experimental import pallas as pl
from jax.experimental.pallas import tpu as pltpu
from jax.experimental.pallas import tpu_sc as plsc

TXT_LEN = 512
IMG_LEN = 256
TOT_LEN = TXT_LEN + IMG_LEN
NUM_HEADS = 16
BATCH = 2
R = 16
NW = 32


def _bucket_luts(anchor):
    eps = jnp.finfo(jnp.float32).eps
    d = jnp.arange(-(TXT_LEN - 1), TXT_LEN, dtype=jnp.int32) + anchor
    n = -d
    r0 = (n < 0).astype(jnp.int32) * 16
    n = jnp.abs(n)
    vl = 8 + (jnp.log(n.astype(jnp.float32) / 8 + eps) / jnp.log(128.0 / 8) * 8).astype(jnp.int32)
    vl = jnp.minimum(vl, 15)
    lut_txt = r0 + jnp.where(n < 8, n, vl)
    lut_txt = jnp.concatenate([lut_txt, jnp.zeros((1,), jnp.int32)])

    dy = (jnp.arange(-15, 16, dtype=jnp.int32) + anchor)[:, None]
    dx = (jnp.arange(-15, 16, dtype=jnp.int32) + anchor)[None, :]
    nx, ny = -dx, -dy
    r = (jnp.logical_and(nx <= 0, ny < 0).astype(jnp.int32) * 192
         + jnp.logical_and(nx < 0, ny >= 0).astype(jnp.int32) * 128
         + jnp.logical_and(nx > 0, ny <= 0).astype(jnp.int32) * 64)
    nx, ny = jnp.abs(nx), jnp.abs(ny)
    vx = 4 + (jnp.log(nx.astype(jnp.float32) / 4 + eps) / jnp.log(20.0 / 4) * 4).astype(jnp.int32)
    vx = jnp.minimum(vx, 7)
    vy = 4 + (jnp.log(ny.astype(jnp.float32) / 4 + eps) / jnp.log(20.0 / 4) * 4).astype(jnp.int32)
    vy = jnp.minimum(vy, 7)
    xx = jnp.where(nx < 4, nx, vx)
    yy = jnp.where(ny < 4, ny, vy)
    lut_img = (r + xx + 8 * yy).reshape(-1)
    lut_img = jnp.concatenate([lut_img, jnp.zeros((63,), jnp.int32)])
    return lut_txt, lut_img


def _sc_fill(txt_pos, img_pos, rel_emb, img_rel_emb, lut_txt, lut_img):
    mesh = plsc.VectorSubcoreMesh(core_axis_name="c", subcore_axis_name="s")

    @functools.partial(
        pl.kernel,
        mesh=mesh,
        compiler_params=pltpu.CompilerParams(needs_layout_passes=False),
        out_type=jax.ShapeDtypeStruct((BATCH, NUM_HEADS, TOT_LEN, TOT_LEN), jnp.float32),
        scratch_types=[
            pltpu.VMEM((TXT_LEN,), jnp.int32),
            pltpu.VMEM((IMG_LEN,), jnp.int32),
            pltpu.VMEM((32,), jnp.float32),
            pltpu.VMEM((256,), jnp.float32),
            pltpu.VMEM((1024,), jnp.int32),
            pltpu.VMEM((1024,), jnp.int32),
            pltpu.VMEM((1024,), jnp.float32),
            pltpu.VMEM((1024,), jnp.float32),
            pltpu.VMEM((2 * R, TOT_LEN), jnp.float32),
            pltpu.SemaphoreType.DMA,
            pltpu.SemaphoreType.DMA,
        ],
    )
    def body(txt_pos_hbm, img_pos_hbm, rel_hbm, img_rel_hbm,
             lut_txt_hbm, lut_img_hbm, out_hbm,
             tpos_v, ipos_v, trow_v, irow_v, ltxt_v, limg_v,
             ttab_v, itab_v, buf_v, sem, sem_in):
        wid = lax.axis_index("s") * 2 + lax.axis_index("c")
        b = wid // NUM_HEADS
        h = wid % NUM_HEADS

        copies = [
            pltpu.async_copy(txt_pos_hbm.at[b], tpos_v, sem_in),
            pltpu.async_copy(img_pos_hbm.at[b], ipos_v, sem_in),
            pltpu.async_copy(rel_hbm.at[h], trow_v, sem_in),
            pltpu.async_copy(img_rel_hbm.at[h], irow_v, sem_in),
            pltpu.async_copy(lut_txt_hbm, ltxt_v, sem_in),
            pltpu.async_copy(lut_img_hbm, limg_v, sem_in),
        ]
        for cp in copies:
            cp.wait()

        def bf16_round(i, ref_v):
            v = ref_v[pl.ds(i * 16, 16)]
            u = plsc.bitcast(v, jnp.uint32)
            bias = jnp.uint32(0x7FFF) + ((u >> 16) & jnp.uint32(1))
            u = (u + bias) & jnp.uint32(0xFFFF0000)
            ref_v[pl.ds(i * 16, 16)] = plsc.bitcast(u, jnp.float32)

        for i in range(2):
            bf16_round(i, trow_v)
        for i in range(16):
            bf16_round(i, irow_v)

        def build(tab_v, lut_v, row_v, quarter):
            idxs = [lut_v[pl.ds((quarter * 16 + i) * 16, 16)] for i in range(16)]
            vals = [plsc.load_gather(row_v, [idx]) for idx in idxs]
            for i in range(16):
                tab_v[pl.ds((quarter * 16 + i) * 16, 16)] = vals[i]

        for q in range(4):
            build(ttab_v, ltxt_v, trow_v, q)
        for q in range(4):
            build(itab_v, limg_v, irow_v, q)

        def conv(i, _):
            p = ipos_v[pl.ds(i * 16, 16)]
            ipos_v[pl.ds(i * 16, 16)] = (p & 15) + 31 * (p >> 4)
            return _
        lax.fori_loop(0, IMG_LEN // 16, conv, None)

        zeros16 = jnp.zeros((16,), jnp.float32)

        def wait_one():
            pltpu.make_async_copy(
                buf_v.at[pl.ds(0, R)],
                out_hbm.at[b, h, pl.ds(0, R)], sem).wait()

        tpj = [tpos_v[pl.ds(c * 16, 16)] + (TXT_LEN - 1) for c in range(TXT_LEN // 16)]

        def txt_block(blk, _):
            off = (blk & 1) * R
            i0 = blk * R

            @pl.when(blk >= 2)
            def _w():
                wait_one()

            @pl.when(blk < 2)
            def _z():
                def zrow(r2, _):
                    for c in range(IMG_LEN // 16):
                        buf_v[off + r2, pl.ds(TXT_LEN + c * 16, 16)] = zeros16
                    return _
                lax.fori_loop(0, R, zrow, None)

            def group(g, _):
                svec = tpos_v[pl.ds(i0 + g * 16, 16)]
                row0 = off + g * 16
                for r2 in range(16):
                    s = svec[r2]
                    for half in range(2):
                        vals = [plsc.load_gather(ttab_v, [tpj[half * 16 + c] - s])
                                for c in range(16)]
                        for c in range(16):
                            buf_v[row0 + r2, pl.ds((half * 16 + c) * 16, 16)] = vals[c]
                return _
            lax.fori_loop(0, R // 16, group, None)
            pltpu.async_copy(buf_v.at[pl.ds(off, R)],
                             out_hbm.at[b, h, pl.ds(i0, R)], sem)
            return _
        lax.fori_loop(0, TXT_LEN // R, txt_block, None)

        ipj = [ipos_v[pl.ds(c * 16, 16)] + 480 for c in range(IMG_LEN // 16)]

        def img_block(blk, _):
            off = (blk & 1) * R
            m0 = blk * R

            wait_one()

            @pl.when(blk < 2)
            def _z():
                def zrow(r2, _):
                    for c in range(TXT_LEN // 16):
                        buf_v[off + r2, pl.ds(c * 16, 16)] = zeros16
                    return _
                lax.fori_loop(0, R, zrow, None)

            def group(g, _):
                svec = ipos_v[pl.ds(m0 + g * 16, 16)]
                row0 = off + g * 16
                for r2 in range(16):
                    s = svec[r2]
                    vals = [plsc.load_gather(itab_v, [ipj[c] - s])
                            for c in range(16)]
                    for c in range(16):
                        buf_v[row0 + r2, pl.ds(TXT_LEN + c * 16, 16)] = vals[c]
                return _
            lax.fori_loop(0, R // 16, group, None)
            pltpu.async_copy(
                buf_v.at[pl.ds(off, R)],
                out_hbm.at[b, h, pl.ds(TXT_LEN + m0, R)], sem)
            return _
        lax.fori_loop(0, IMG_LEN // R, img_block, None)
        wait_one()
        wait_one()

    return body(txt_pos, img_pos, rel_emb, img_rel_emb, lut_txt, lut_img)


def kernel(txt_position_ids, img_position_ids, rel_embedding, image_rel_embedding):
    anchor = (txt_position_ids[0, 0] * 0).astype(jnp.int32)
    lut_txt, lut_img = _bucket_luts(anchor)
    return _sc_fill(
        txt_position_ids.astype(jnp.int32),
        img_position_ids.astype(jnp.int32),
        rel_embedding.astype(jnp.float32),
        image_rel_embedding.astype(jnp.float32),
        lut_txt, lut_img)


# speedup vs baseline: 1.3540x; 1.1359x over previous
import functools

import jax
import jax.numpy as jnp
from jax import lax
from jax.experimental import pallas as pl
from jax.experimental.pallas import tpu as pltpu
from jax.experimental.pallas import tpu_sc as plsc

TXT_LEN = 512
IMG_LEN = 256
TOT_LEN = TXT_LEN + IMG_LEN
NUM_HEADS = 16
BATCH = 2
R = 8
NW = 32


def _bucket_luts(anchor):
    eps = jnp.finfo(jnp.float32).eps
    d = jnp.arange(-(TXT_LEN - 1), TXT_LEN, dtype=jnp.int32) + anchor
    n = -d
    r0 = (n < 0).astype(jnp.int32) * 16
    n = jnp.abs(n)
    vl = 8 + (jnp.log(n.astype(jnp.float32) / 8 + eps) / jnp.log(128.0 / 8) * 8).astype(jnp.int32)
    vl = jnp.minimum(vl, 15)
    lut_txt = r0 + jnp.where(n < 8, n, vl)
    lut_txt = jnp.concatenate([lut_txt, jnp.zeros((1,), jnp.int32)])

    dy = (jnp.arange(-15, 16, dtype=jnp.int32) + anchor)[:, None]
    dx = (jnp.arange(-15, 16, dtype=jnp.int32) + anchor)[None, :]
    nx, ny = -dx, -dy
    r = (jnp.logical_and(nx <= 0, ny < 0).astype(jnp.int32) * 192
         + jnp.logical_and(nx < 0, ny >= 0).astype(jnp.int32) * 128
         + jnp.logical_and(nx > 0, ny <= 0).astype(jnp.int32) * 64)
    nx, ny = jnp.abs(nx), jnp.abs(ny)
    vx = 4 + (jnp.log(nx.astype(jnp.float32) / 4 + eps) / jnp.log(20.0 / 4) * 4).astype(jnp.int32)
    vx = jnp.minimum(vx, 7)
    vy = 4 + (jnp.log(ny.astype(jnp.float32) / 4 + eps) / jnp.log(20.0 / 4) * 4).astype(jnp.int32)
    vy = jnp.minimum(vy, 7)
    xx = jnp.where(nx < 4, nx, vx)
    yy = jnp.where(ny < 4, ny, vy)
    lut_img = (r + xx + 8 * yy).reshape(-1)
    lut_img = jnp.concatenate([lut_img, jnp.zeros((63,), jnp.int32)])
    return lut_txt, lut_img


def _sc_fill(txt_pos, img_pos, rel_emb, img_rel_emb, lut_txt, lut_img):
    mesh = plsc.VectorSubcoreMesh(core_axis_name="c", subcore_axis_name="s")

    @functools.partial(
        pl.kernel,
        mesh=mesh,
        compiler_params=pltpu.CompilerParams(needs_layout_passes=False),
        out_type=jax.ShapeDtypeStruct((BATCH, NUM_HEADS, TOT_LEN, TOT_LEN), jnp.float32),
        scratch_types=[
            pltpu.VMEM((TXT_LEN,), jnp.int32),
            pltpu.VMEM((IMG_LEN,), jnp.int32),
            pltpu.VMEM((32,), jnp.float32),
            pltpu.VMEM((256,), jnp.float32),
            pltpu.VMEM((1024,), jnp.int32),
            pltpu.VMEM((1024,), jnp.int32),
            pltpu.VMEM((1024,), jnp.float32),
            pltpu.VMEM((1024,), jnp.float32),
            pltpu.VMEM((2 * R, TOT_LEN), jnp.float32),
            pltpu.SemaphoreType.DMA,
            pltpu.SemaphoreType.DMA,
        ],
    )
    def body(txt_pos_hbm, img_pos_hbm, rel_hbm, img_rel_hbm,
             lut_txt_hbm, lut_img_hbm, out_hbm,
             tpos_v, ipos_v, trow_v, irow_v, ltxt_v, limg_v,
             ttab_v, itab_v, buf_v, sem, sem_in):
        wid = lax.axis_index("s") * 2 + lax.axis_index("c")
        b = wid // NUM_HEADS
        h = wid % NUM_HEADS

        copies = [
            pltpu.async_copy(txt_pos_hbm.at[b], tpos_v, sem_in),
            pltpu.async_copy(img_pos_hbm.at[b], ipos_v, sem_in),
            pltpu.async_copy(rel_hbm.at[h], trow_v, sem_in),
            pltpu.async_copy(img_rel_hbm.at[h], irow_v, sem_in),
            pltpu.async_copy(lut_txt_hbm, ltxt_v, sem_in),
            pltpu.async_copy(lut_img_hbm, limg_v, sem_in),
        ]
        for cp in copies:
            cp.wait()

        def bf16_round(i, ref_v):
            v = ref_v[pl.ds(i * 16, 16)]
            u = plsc.bitcast(v, jnp.uint32)
            bias = jnp.uint32(0x7FFF) + ((u >> 16) & jnp.uint32(1))
            u = (u + bias) & jnp.uint32(0xFFFF0000)
            ref_v[pl.ds(i * 16, 16)] = plsc.bitcast(u, jnp.float32)

        for i in range(2):
            bf16_round(i, trow_v)
        for i in range(16):
            bf16_round(i, irow_v)

        def build(tab_v, lut_v, row_v, quarter):
            idxs = [lut_v[pl.ds((quarter * 16 + i) * 16, 16)] for i in range(16)]
            vals = [plsc.load_gather(row_v, [idx]) for idx in idxs]
            for i in range(16):
                tab_v[pl.ds((quarter * 16 + i) * 16, 16)] = vals[i]

        for q in range(4):
            build(ttab_v, ltxt_v, trow_v, q)
        for q in range(4):
            build(itab_v, limg_v, irow_v, q)

        def conv(i, _):
            p = ipos_v[pl.ds(i * 16, 16)]
            ipos_v[pl.ds(i * 16, 16)] = (p & 15) + 31 * (p >> 4)
            return _
        lax.fori_loop(0, IMG_LEN // 16, conv, None)

        zeros16 = jnp.zeros((16,), jnp.float32)

        def wait_one():
            pltpu.make_async_copy(
                buf_v.at[pl.ds(0, R)],
                out_hbm.at[b, h, pl.ds(0, R)], sem).wait()

        tpj = [tpos_v[pl.ds(c * 16, 16)] + (TXT_LEN - 1) for c in range(TXT_LEN // 16)]

        def txt_block(blk, _):
            off = (blk & 1) * R
            i0 = blk * R

            @pl.when(blk >= 2)
            def _w():
                wait_one()

            @pl.when(blk < 2)
            def _z():
                def zrow(r2, _):
                    for c in range(IMG_LEN // 16):
                        buf_v[off + r2, pl.ds(TXT_LEN + c * 16, 16)] = zeros16
                    return _
                lax.fori_loop(0, R, zrow, None)

            def group(g, _):
                svec = tpos_v[pl.ds(i0 + g * 16, 16)]
                row0 = off + g * 16
                for r2 in range(16):
                    s = svec[r2]
                    for half in range(2):
                        vals = [plsc.load_gather(ttab_v, [tpj[half * 16 + c] - s])
                                for c in range(16)]
                        for c in range(16):
                            buf_v[row0 + r2, pl.ds((half * 16 + c) * 16, 16)] = vals[c]
                return _
            lax.fori_loop(0, R // 16, group, None)
            pltpu.async_copy(buf_v.at[pl.ds(off, R)],
                             out_hbm.at[b, h, pl.ds(i0, R)], sem)
            return _
        lax.fori_loop(0, TXT_LEN // R, txt_block, None)

        ipj = [ipos_v[pl.ds(c * 16, 16)] + 480 for c in range(IMG_LEN // 16)]

        def img_block(blk, _):
            off = (blk & 1) * R
            m0 = blk * R

            wait_one()

            @pl.when(blk < 2)
            def _z():
                def zrow(r2, _):
                    for c in range(TXT_LEN // 16):
                        buf_v[off + r2, pl.ds(c * 16, 16)] = zeros16
                    return _
                lax.fori_loop(0, R, zrow, None)

            def group(g, _):
                svec = ipos_v[pl.ds(m0 + g * 16, 16)]
                row0 = off + g * 16
                for r2 in range(16):
                    s = svec[r2]
                    vals = [plsc.load_gather(itab_v, [ipj[c] - s])
                            for c in range(16)]
                    for c in range(16):
                        buf_v[row0 + r2, pl.ds(TXT_LEN + c * 16, 16)] = vals[c]
                return _
            lax.fori_loop(0, R // 16, group, None)
            pltpu.async_copy(
                buf_v.at[pl.ds(off, R)],
                out_hbm.at[b, h, pl.ds(TXT_LEN + m0, R)], sem)
            return _
        lax.fori_loop(0, IMG_LEN // R, img_block, None)
        wait_one()
        wait_one()

    return body(txt_pos, img_pos, rel_emb, img_rel_emb, lut_txt, lut_img)


def kernel(txt_position_ids, img_position_ids, rel_embedding, image_rel_embedding):
    anchor = (txt_position_ids[0, 0] * 0).astype(jnp.int32)
    lut_txt, lut_img = _bucket_luts(anchor)
    return _sc_fill(
        txt_position_ids.astype(jnp.int32),
        img_position_ids.astype(jnp.int32),
        rel_embedding.astype(jnp.float32),
        image_rel_embedding.astype(jnp.float32),
        lut_txt, lut_img)
